# baseline (device time: 55498 ns/iter reference)
import jax
import jax.numpy as jnp
from jax import lax
from jax.experimental import pallas as pl
from jax.experimental.pallas import tpu as pltpu


def kernel(A, B):
    m, k = A.shape
    k2, n = B.shape
    assert k == k2

    def body(a_ref, b_ref, out_ref, comm_ref, send_sem, recv_sem):
        my_x = lax.axis_index("x")
        my_y = lax.axis_index("y")
        nbr = (1 - my_x, my_y)

        barrier_sem = pltpu.get_barrier_semaphore()
        pl.semaphore_signal(
            barrier_sem, inc=1, device_id=nbr,
            device_id_type=pl.DeviceIdType.MESH,
        )
        pl.semaphore_wait(barrier_sem, 1)

        out_ref[...] = jnp.dot(
            a_ref[...], b_ref[...], preferred_element_type=jnp.float32
        )

        rdma = pltpu.make_async_remote_copy(
            src_ref=out_ref,
            dst_ref=comm_ref,
            send_sem=send_sem,
            recv_sem=recv_sem,
            device_id=nbr,
            device_id_type=pl.DeviceIdType.MESH,
        )
        rdma.start()
        rdma.wait()

        out_ref[...] = out_ref[...] + comm_ref[...]

    return pl.pallas_call(
        body,
        out_shape=jax.ShapeDtypeStruct((m, n), jnp.float32),
        in_specs=[
            pl.BlockSpec(memory_space=pltpu.VMEM),
            pl.BlockSpec(memory_space=pltpu.VMEM),
        ],
        out_specs=pl.BlockSpec(memory_space=pltpu.VMEM),
        scratch_shapes=[
            pltpu.VMEM((m, n), jnp.float32),
            pltpu.SemaphoreType.DMA,
            pltpu.SemaphoreType.DMA,
        ],
        compiler_params=pltpu.CompilerParams(collective_id=0),
    )(A, B)


# device time: 37296 ns/iter; 1.4880x vs baseline; 1.4880x over previous
import jax
import jax.numpy as jnp
from jax import lax
from jax.experimental import pallas as pl
from jax.experimental.pallas import tpu as pltpu

N_CHUNK = 8
COMPUTE_CHUNKS = 2


def kernel(A, B):
    m, k = A.shape
    k2, n = B.shape
    assert k == k2
    half_rows = m // 2
    ch = half_rows // N_CHUNK
    cc_rows = half_rows // COMPUTE_CHUNKS
    per_cc = N_CHUNK // COMPUTE_CHUNKS

    def body(a_ref, b_ref, out_ref, xrecv, xs_sem, xr_sem, ys_sem, yr_sem):
        my_x = lax.axis_index("x")
        my_y = lax.axis_index("y")
        xnbr = (1 - my_x, my_y)
        ynbr = (my_x, 1 - my_y)
        half = my_y * half_rows
        other = (1 - my_y) * half_rows

        barrier_sem = pltpu.get_barrier_semaphore()
        for nbr in (xnbr, ynbr):
            pl.semaphore_signal(
                barrier_sem, inc=1, device_id=nbr,
                device_id_type=pl.DeviceIdType.MESH,
            )
        pl.semaphore_wait(barrier_sem, 2)

        def x_rdma(c):
            return pltpu.make_async_remote_copy(
                src_ref=out_ref.at[pl.ds(half + c * ch, ch), :],
                dst_ref=xrecv.at[pl.ds(c * ch, ch), :],
                send_sem=xs_sem.at[c],
                recv_sem=xr_sem.at[c],
                device_id=xnbr,
                device_id_type=pl.DeviceIdType.MESH,
            )

        def y_rdma(c):
            return pltpu.make_async_remote_copy(
                src_ref=out_ref.at[pl.ds(half + c * ch, ch), :],
                dst_ref=out_ref.at[pl.ds(half + c * ch, ch), :],
                send_sem=ys_sem.at[c],
                recv_sem=yr_sem.at[c],
                device_id=ynbr,
                device_id_type=pl.DeviceIdType.MESH,
            )

        for cc in range(COMPUTE_CHUNKS):
            r0 = half + cc * cc_rows
            out_ref[pl.ds(r0, cc_rows), :] = jnp.dot(
                a_ref[pl.ds(r0, cc_rows), :],
                b_ref[...],
                preferred_element_type=jnp.float32,
            )
            for j in range(per_cc):
                x_rdma(cc * per_cc + j).start()

        for c in range(N_CHUNK):
            r = x_rdma(c)
            r.wait()
            out_ref[pl.ds(half + c * ch, ch), :] = (
                out_ref[pl.ds(half + c * ch, ch), :]
                + xrecv[pl.ds(c * ch, ch), :]
            )
            y_rdma(c).start()

        for c in range(N_CHUNK):
            y_rdma(c).wait_send()
            y_wait = pltpu.make_async_remote_copy(
                src_ref=out_ref.at[pl.ds(half + c * ch, ch), :],
                dst_ref=out_ref.at[pl.ds(other + c * ch, ch), :],
                send_sem=ys_sem.at[c],
                recv_sem=yr_sem.at[c],
                device_id=ynbr,
                device_id_type=pl.DeviceIdType.MESH,
            )
            y_wait.wait_recv()

    return pl.pallas_call(
        body,
        out_shape=jax.ShapeDtypeStruct((m, n), jnp.float32),
        in_specs=[
            pl.BlockSpec(memory_space=pltpu.VMEM),
            pl.BlockSpec(memory_space=pltpu.VMEM),
        ],
        out_specs=pl.BlockSpec(memory_space=pltpu.VMEM),
        scratch_shapes=[
            pltpu.VMEM((half_rows, n), jnp.float32),
            pltpu.SemaphoreType.DMA((N_CHUNK,)),
            pltpu.SemaphoreType.DMA((N_CHUNK,)),
            pltpu.SemaphoreType.DMA((N_CHUNK,)),
            pltpu.SemaphoreType.DMA((N_CHUNK,)),
        ],
        compiler_params=pltpu.CompilerParams(collective_id=0),
    )(A, B)


# device time: 35935 ns/iter; 1.5444x vs baseline; 1.0379x over previous
import jax
import jax.numpy as jnp
from jax import lax
from jax.experimental import pallas as pl
from jax.experimental.pallas import tpu as pltpu

N_CHUNK = 16
COMPUTE_CHUNKS = 4


def kernel(A, B):
    m, k = A.shape
    k2, n = B.shape
    assert k == k2
    half_rows = m // 2
    ch = half_rows // N_CHUNK
    cc_rows = half_rows // COMPUTE_CHUNKS
    per_cc = N_CHUNK // COMPUTE_CHUNKS

    def body(a_ref, b_ref, out_ref, xrecv, xs_sem, xr_sem, ys_sem, yr_sem):
        my_x = lax.axis_index("x")
        my_y = lax.axis_index("y")
        xnbr = (1 - my_x, my_y)
        ynbr = (my_x, 1 - my_y)
        half = my_y * half_rows
        other = (1 - my_y) * half_rows

        barrier_sem = pltpu.get_barrier_semaphore()
        for nbr in (xnbr, ynbr):
            pl.semaphore_signal(
                barrier_sem, inc=1, device_id=nbr,
                device_id_type=pl.DeviceIdType.MESH,
            )
        pl.semaphore_wait(barrier_sem, 2)

        def x_rdma(c):
            return pltpu.make_async_remote_copy(
                src_ref=out_ref.at[pl.ds(half + c * ch, ch), :],
                dst_ref=xrecv.at[pl.ds(c * ch, ch), :],
                send_sem=xs_sem.at[c],
                recv_sem=xr_sem.at[c],
                device_id=xnbr,
                device_id_type=pl.DeviceIdType.MESH,
            )

        def y_rdma(c):
            return pltpu.make_async_remote_copy(
                src_ref=out_ref.at[pl.ds(half + c * ch, ch), :],
                dst_ref=out_ref.at[pl.ds(half + c * ch, ch), :],
                send_sem=ys_sem.at[c],
                recv_sem=yr_sem.at[c],
                device_id=ynbr,
                device_id_type=pl.DeviceIdType.MESH,
            )

        for cc in range(COMPUTE_CHUNKS):
            r0 = half + cc * cc_rows
            out_ref[pl.ds(r0, cc_rows), :] = jnp.dot(
                a_ref[pl.ds(r0, cc_rows), :],
                b_ref[...],
                preferred_element_type=jnp.float32,
            )
            for j in range(per_cc):
                x_rdma(cc * per_cc + j).start()

        for c in range(N_CHUNK):
            r = x_rdma(c)
            r.wait()
            out_ref[pl.ds(half + c * ch, ch), :] = (
                out_ref[pl.ds(half + c * ch, ch), :]
                + xrecv[pl.ds(c * ch, ch), :]
            )
            y_rdma(c).start()

        for c in range(N_CHUNK):
            y_rdma(c).wait_send()
            y_wait = pltpu.make_async_remote_copy(
                src_ref=out_ref.at[pl.ds(half + c * ch, ch), :],
                dst_ref=out_ref.at[pl.ds(other + c * ch, ch), :],
                send_sem=ys_sem.at[c],
                recv_sem=yr_sem.at[c],
                device_id=ynbr,
                device_id_type=pl.DeviceIdType.MESH,
            )
            y_wait.wait_recv()

    return pl.pallas_call(
        body,
        out_shape=jax.ShapeDtypeStruct((m, n), jnp.float32),
        in_specs=[
            pl.BlockSpec(memory_space=pltpu.VMEM),
            pl.BlockSpec(memory_space=pltpu.VMEM),
        ],
        out_specs=pl.BlockSpec(memory_space=pltpu.VMEM),
        scratch_shapes=[
            pltpu.VMEM((half_rows, n), jnp.float32),
            pltpu.SemaphoreType.DMA((N_CHUNK,)),
            pltpu.SemaphoreType.DMA((N_CHUNK,)),
            pltpu.SemaphoreType.DMA((N_CHUNK,)),
            pltpu.SemaphoreType.DMA((N_CHUNK,)),
        ],
        compiler_params=pltpu.CompilerParams(collective_id=0),
    )(A, B)


# device time: 35586 ns/iter; 1.5595x vs baseline; 1.0098x over previous
import jax
import jax.numpy as jnp
from jax import lax
from jax.experimental import pallas as pl
from jax.experimental.pallas import tpu as pltpu

N_CHUNK = 32
COMPUTE_CHUNKS = 4


def kernel(A, B):
    m, k = A.shape
    k2, n = B.shape
    assert k == k2
    half_rows = m // 2
    ch = half_rows // N_CHUNK
    cc_rows = half_rows // COMPUTE_CHUNKS
    per_cc = N_CHUNK // COMPUTE_CHUNKS

    def body(a_ref, b_ref, out_ref, xrecv, xs_sem, xr_sem, ys_sem, yr_sem):
        my_x = lax.axis_index("x")
        my_y = lax.axis_index("y")
        xnbr = (1 - my_x, my_y)
        ynbr = (my_x, 1 - my_y)
        half = my_y * half_rows
        other = (1 - my_y) * half_rows

        barrier_sem = pltpu.get_barrier_semaphore()
        for nbr in (xnbr, ynbr):
            pl.semaphore_signal(
                barrier_sem, inc=1, device_id=nbr,
                device_id_type=pl.DeviceIdType.MESH,
            )

        def x_rdma(c):
            return pltpu.make_async_remote_copy(
                src_ref=out_ref.at[pl.ds(half + c * ch, ch), :],
                dst_ref=xrecv.at[pl.ds(c * ch, ch), :],
                send_sem=xs_sem.at[c],
                recv_sem=xr_sem.at[c],
                device_id=xnbr,
                device_id_type=pl.DeviceIdType.MESH,
            )

        def y_rdma(c):
            return pltpu.make_async_remote_copy(
                src_ref=out_ref.at[pl.ds(half + c * ch, ch), :],
                dst_ref=out_ref.at[pl.ds(half + c * ch, ch), :],
                send_sem=ys_sem.at[c],
                recv_sem=yr_sem.at[c],
                device_id=ynbr,
                device_id_type=pl.DeviceIdType.MESH,
            )

        for cc in range(COMPUTE_CHUNKS):
            r0 = half + cc * cc_rows
            out_ref[pl.ds(r0, cc_rows), :] = jnp.dot(
                a_ref[pl.ds(r0, cc_rows), :],
                b_ref[...],
                preferred_element_type=jnp.float32,
            )
            if cc == 0:
                pl.semaphore_wait(barrier_sem, 2)
            for j in range(per_cc):
                x_rdma(cc * per_cc + j).start()

        for c in range(N_CHUNK):
            r = x_rdma(c)
            r.wait()
            out_ref[pl.ds(half + c * ch, ch), :] = (
                out_ref[pl.ds(half + c * ch, ch), :]
                + xrecv[pl.ds(c * ch, ch), :]
            )
            y_rdma(c).start()

        for c in range(N_CHUNK):
            y_rdma(c).wait_send()
            y_wait = pltpu.make_async_remote_copy(
                src_ref=out_ref.at[pl.ds(half + c * ch, ch), :],
                dst_ref=out_ref.at[pl.ds(other + c * ch, ch), :],
                send_sem=ys_sem.at[c],
                recv_sem=yr_sem.at[c],
                device_id=ynbr,
                device_id_type=pl.DeviceIdType.MESH,
            )
            y_wait.wait_recv()

    return pl.pallas_call(
        body,
        out_shape=jax.ShapeDtypeStruct((m, n), jnp.float32),
        in_specs=[
            pl.BlockSpec(memory_space=pltpu.VMEM),
            pl.BlockSpec(memory_space=pltpu.VMEM),
        ],
        out_specs=pl.BlockSpec(memory_space=pltpu.VMEM),
        scratch_shapes=[
            pltpu.VMEM((half_rows, n), jnp.float32),
            pltpu.SemaphoreType.DMA((N_CHUNK,)),
            pltpu.SemaphoreType.DMA((N_CHUNK,)),
            pltpu.SemaphoreType.DMA((N_CHUNK,)),
            pltpu.SemaphoreType.DMA((N_CHUNK,)),
        ],
        compiler_params=pltpu.CompilerParams(collective_id=0),
    )(A, B)
